# baseline (speedup 1.0000x reference)
import functools

import jax
import jax.numpy as jnp
from jax import lax
from jax.experimental import pallas as pl
from jax.experimental.pallas import tpu as pltpu
from jax.experimental.pallas import tpu_sc as plsc

NC = 2
NS = 16
NW = NC * NS
LANES = 16


def _make_kernel(B, S, V, D):
    rows_total = B * S
    assert rows_total % NW == 0
    rows_per_w = rows_total // NW
    assert rows_per_w % S == 0
    seqs_per_w = rows_per_w // S
    half = S // 2
    assert S % 2 == 0 and D % LANES == 0

    mesh = plsc.VectorSubcoreMesh(
        core_axis_name="c", subcore_axis_name="s",
        num_cores=NC, num_subcores=NS)

    @functools.partial(
        pl.kernel,
        out_type=jax.ShapeDtypeStruct((rows_total, D), jnp.float32),
        mesh=mesh,
        scratch_types=[
            pltpu.VMEM((2, half), jnp.int32),
            pltpu.VMEM((S, D), jnp.float32),
            pltpu.VMEM((S, D), jnp.float32),
            pltpu.SemaphoreType.DMA,
        ],
        compiler_params=pltpu.CompilerParams(use_tc_tiling_on_sc=False),
    )
    def k(x2_hbm, tok_hbm, pos_hbm, out_hbm, idx_v, rows_v, pos_v, sem):
        wid = lax.axis_index("s") * NC + lax.axis_index("c")
        pltpu.sync_copy(pos_hbm, pos_v)

        @pl.loop(0, seqs_per_w)
        def _seq(cidx):
            base = wid * rows_per_w + cidx * S
            pltpu.sync_copy(x2_hbm.at[pl.ds(wid * seqs_per_w * 2 + cidx * 2, 2)],
                            idx_v)
            cp0 = pltpu.async_copy(tok_hbm.at[idx_v.at[0]],
                                   rows_v.at[pl.ds(0, half)], sem)
            cp1 = pltpu.async_copy(tok_hbm.at[idx_v.at[1]],
                                   rows_v.at[pl.ds(half, half)], sem)
            cp0.wait()
            cp1.wait()

            @pl.loop(0, S)
            def _row(r):
                for j in range(D // LANES):
                    sl = pl.ds(j * LANES, LANES)
                    rows_v[r, sl] = rows_v[r, sl] + pos_v[r, sl]

            pltpu.sync_copy(rows_v, out_hbm.at[pl.ds(base, S)])

    return k


def kernel(x, token_table, pos_table):
    B, S = x.shape
    V, D = token_table.shape
    x2 = x.reshape(B * S // (S // 2), S // 2).astype(jnp.int32)
    k = _make_kernel(B, S, V, D)
    out = k(x2, token_table, pos_table)
    return out.reshape(B, S, D)
